# baseline (speedup 1.0000x reference)
import functools

import jax
import jax.numpy as jnp
from jax import lax
from jax.experimental import pallas as pl
from jax.experimental.pallas import tpu as pltpu
from jax.experimental.pallas import tpu_sc as plsc

_BATCH = 4096
_SEQ = 200
_D = 64
_TOTAL = _BATCH * _SEQ
_V = 1000000

_NC = 2
_NS = 16
_NW = _NC * _NS
_PER_W = _TOTAL // _NW

_CONV_BLK = 512
_CONV_GRID = 977
_H = _CONV_BLK * _CONV_GRID

_mesh = plsc.VectorSubcoreMesh(core_axis_name="c", subcore_axis_name="s")


def _iota16(mult):
    return lax.broadcasted_iota(jnp.int32, (16,), 0) * mult


def _conv_body(x1_ref, x2_ref, o_ref):
    o_ref[:, 0:_D] = x1_ref[...].T
    o_ref[:, _D:128] = x2_ref[...].T


_conv = pl.pallas_call(
    _conv_body,
    out_shape=jax.ShapeDtypeStruct((_H, 128), jnp.float32),
    grid=(_CONV_GRID,),
    in_specs=[
        pl.BlockSpec((_D, _CONV_BLK), lambda i: (0, i)),
        pl.BlockSpec((_D, _CONV_BLK), lambda i: (0, i + _CONV_GRID)),
    ],
    out_specs=pl.BlockSpec((_CONV_BLK, 128), lambda i: (i, 0)),
)


def _gather_body(tab_hbm, idx_hbm, out_hbm, idx_v, pairs_v, rows_v, tbuf_v,
                 gs0, gs1, ss0, ss1):
    gsems = (gs0, gs1)
    ssems = (ss0, ss1)
    wid = lax.axis_index("s") * _NC + lax.axis_index("c")

    pltpu.sync_copy(idx_hbm.at[pl.ds(wid * _PER_W, _PER_W)], idx_v)

    iota200 = _iota16(_SEQ)
    iota1 = _iota16(1)

    def mkpairs(s, carry):
        for j in range(8):
            addr = iota200 + (j * 16 * _SEQ + s)
            v = plsc.load_gather(idx_v, [addr])
            pair = v - jnp.where(v >= _H, _H, 0)
            pairs_v[pl.ds(s * 128 + 16 * j, 16)] = pair
        return carry

    lax.fori_loop(0, _SEQ, mkpairs, 0)

    def fire_g(s, b):
        pltpu.async_copy(
            tab_hbm.at[pairs_v.at[pl.ds(s * 128, 128)]], rows_v.at[b], gsems[b]
        )

    def wait_g(b):
        pltpu.make_async_copy(
            tab_hbm.at[pairs_v.at[pl.ds(0, 128)]], rows_v.at[b], gsems[b]
        ).wait()

    def transpose(s, b):
        def jbody(j, carry):
            oaddr = iota200 + (j * 16 * _SEQ + s)
            ov = plsc.load_gather(idx_v, [oaddr])
            h64 = jnp.where(ov >= _H, _D, 0)
            rowidx = iota1 + (16 * j)
            off = pl.multiple_of(16 * j, 16)
            for c in range(_D):
                val = plsc.load_gather(rows_v.at[b], [rowidx, h64 + c])
                tbuf_v[b, c // 8, c % 8, pl.ds(off, 16)] = val
            return carry

        lax.fori_loop(0, 8, jbody, 0)

    def fire_s(s, b):
        for c8 in range(8):
            pltpu.async_copy(
                tbuf_v.at[b, c8], out_hbm.at[s].at[c8].at[wid], ssems[b]
            )

    def wait_s(b):
        for c8 in range(8):
            pltpu.make_async_copy(
                tbuf_v.at[b, c8], out_hbm.at[0].at[c8].at[wid], ssems[b]
            ).wait()

    fire_g(0, 0)
    fire_g(1, 1)

    def lap(k, carry):
        for b in (0, 1):
            s = 2 * k + b
            wait_g(b)

            @pl.when(s >= 2)
            def _():
                wait_s(b)

            transpose(s, b)
            fire_s(s, b)

            @pl.when(s + 2 < _SEQ)
            def _():
                fire_g(s + 2, b)

        return carry

    lax.fori_loop(0, _SEQ // 2, lap, 0)
    wait_s(0)
    wait_s(1)


_gather = functools.partial(
    pl.kernel,
    out_type=jax.ShapeDtypeStruct((_SEQ, 8, _NW, 8, 128), jnp.float32),
    mesh=_mesh,
    scratch_types=[
        pltpu.VMEM((_PER_W,), jnp.int32),
        pltpu.VMEM((_PER_W,), jnp.int32),
        pltpu.VMEM((2, 128, 128), jnp.float32),
        pltpu.VMEM((2, 8, 8, 128), jnp.float32),
        pltpu.SemaphoreType.DMA,
        pltpu.SemaphoreType.DMA,
        pltpu.SemaphoreType.DMA,
        pltpu.SemaphoreType.DMA,
    ],
    compiler_params=pltpu.CompilerParams(
        use_tc_tiling_on_sc=False, needs_layout_passes=False
    ),
)(_gather_body)


@jax.jit
def kernel(inputs, table):
    vt = table.T
    tab2 = _conv(vt, vt)
    idx_flat = inputs.reshape(-1).astype(jnp.int32)
    out5 = _gather(tab2, idx_flat)
    return jnp.transpose(out5, (2, 4, 0, 1, 3)).reshape(_BATCH, _SEQ, _D)
